# baseline (speedup 1.0000x reference)
import functools

import jax
import jax.numpy as jnp
from jax import lax
from jax.experimental import pallas as pl
from jax.experimental.pallas import tpu as pltpu
from jax.experimental.pallas import tpu_sc as plsc

N_USERS = 25000
N_ITEMS = 25000
D = 64
E = 800000
FW = 0.3

HALF_PAD = 25600
NPAD = 2 * HALF_PAD
TILES = 16
CHUNK = 128
KCH = 391
E_PAD = TILES * KCH * CHUNK
ROWS_PER_TILE = HALF_PAD // TILES
ZB = 32
NZB = ROWS_PER_TILE // ZB

BLK = 1000


def _sc_propagate(cur, edges, vals):
    mesh = plsc.VectorSubcoreMesh(core_axis_name="c", subcore_axis_name="s")

    @functools.partial(
        pl.kernel,
        out_type=jax.ShapeDtypeStruct((NPAD, D), jnp.float32),
        mesh=mesh,
        compiler_params=pltpu.CompilerParams(use_tc_tiling_on_sc=False),
        scratch_types=[
            pltpu.VMEM_SHARED((HALF_PAD, D), jnp.float32),
            pltpu.VMEM((4, 2, CHUNK), jnp.int32),
            pltpu.VMEM((4, CHUNK), jnp.float32),
            pltpu.VMEM((3, CHUNK, D), jnp.float32),
            pltpu.VMEM((ZB, D), jnp.float32),
            pltpu.SemaphoreType.DMA((4,)),
            pltpu.SemaphoreType.DMA((4,)),
            pltpu.SemaphoreType.DMA((3,)),
            pltpu.SemaphoreType.DMA((3,)),
            pltpu.SemaphoreType.DMA,
        ],
    )
    def k(cur_hbm, edges_hbm, vals_hbm, out_hbm,
          acc, ebuf, vbuf, rows, zblk, sem_e, sem_v, sem_g, sem_s, sem_z):
        c = lax.axis_index("c")
        s = lax.axis_index("s")
        lo = c * HALF_PAD
        rbase = s * ROWS_PER_TILE
        base_ch = s * KCH

        def zz(i, carry):
            for j in range(D // 16):
                zblk[i, pl.ds(j * 16, 16)] = jnp.zeros((16,), jnp.float32)
            return carry
        lax.fori_loop(0, ZB, zz, 0)

        def zacc(b, carry):
            pltpu.async_copy(zblk, acc.at[pl.ds(rbase + b * ZB, ZB)], sem_z)
            return carry
        lax.fori_loop(0, NZB, zacc, 0)

        def zdrain(b, carry):
            pltpu.make_async_copy(
                zblk, acc.at[pl.ds(rbase + b * ZB, ZB)], sem_z).wait()
            return carry
        lax.fori_loop(0, NZB, zdrain, 0)
        plsc.subcore_barrier()

        def in_desc(kk):
            t = lax.rem(kk, 4)
            return pltpu.make_async_copy(
                edges_hbm.at[c, base_ch + kk], ebuf.at[t], sem_e.at[t])

        def val_desc(kk):
            t = lax.rem(kk, 4)
            return pltpu.make_async_copy(
                vals_hbm.at[c, base_ch + kk], vbuf.at[t], sem_v.at[t])

        def gather_desc(kk):
            t = lax.rem(kk, 4)
            b = lax.rem(kk, 3)
            return pltpu.make_async_copy(
                cur_hbm.at[ebuf.at[t, 0]], rows.at[b], sem_g.at[b])

        def scatter_desc(kk):
            t = lax.rem(kk, 4)
            b = lax.rem(kk, 3)
            return pltpu.make_async_copy(
                rows.at[b], acc.at[ebuf.at[t, 1]], sem_s.at[b])

        dnums = lax.GatherDimensionNumbers(
            offset_dims=(), collapsed_slice_dims=(0,), start_index_map=(0,))

        for j in range(2):
            in_desc(j).start()
            val_desc(j).start()
        in_desc(0).wait()
        gather_desc(0).start()

        def chunk_body(kk, carry):
            t = lax.rem(kk, 4)
            b = lax.rem(kk, 3)

            @pl.when(kk >= 2)
            def _():
                scatter_desc(kk - 2).wait()

            @pl.when(kk + 1 < KCH)
            def _():
                in_desc(kk + 1).wait()
                gather_desc(kk + 1).start()

            @pl.when(kk + 2 < KCH)
            def _():
                in_desc(kk + 2).start()
                val_desc(kk + 2).start()
            val_desc(kk).wait()
            gather_desc(kk).wait()

            @plsc.parallel_loop(0, CHUNK // 16, unroll=2)
            def scale_group(g):
                vbase = vbuf[t, pl.ds(g * 16, 16)]
                for i in range(16):
                    v16 = lax.gather(
                        vbase, jnp.full((16, 1), i, jnp.int32), dnums, (1,),
                        mode=lax.GatherScatterMode.PROMISE_IN_BOUNDS)
                    e = g * 16 + i
                    for j in range(D // 16):
                        sl = pl.ds(j * 16, 16)
                        rows[b, e, sl] = rows[b, e, sl] * v16

            scatter_desc(kk).start(add=True)
            return carry
        lax.fori_loop(0, KCH, chunk_body, 0)
        scatter_desc(KCH - 2).wait()
        scatter_desc(KCH - 1).wait()
        plsc.subcore_barrier()

        def wb(b, carry):
            st = rbase + b * ZB
            pltpu.async_copy(
                acc.at[pl.ds(st, ZB)], out_hbm.at[pl.ds(lo + st, ZB)], sem_z)
            return carry
        lax.fori_loop(0, NZB, wb, 0)

        def wb_drain(b, carry):
            st = rbase + b * ZB
            pltpu.make_async_copy(
                acc.at[pl.ds(st, ZB)], out_hbm.at[pl.ds(lo + st, ZB)],
                sem_z).wait()
            return carry
        lax.fori_loop(0, NZB, wb_drain, 0)

    return k(cur, edges, vals)


def _mm_stats(x, w):
    r, f = x.shape
    h = w.shape[1]
    grid = r // BLK

    def kern(x_ref, w_ref, p_ref, s_ref, q_ref):
        p = jnp.dot(x_ref[...], w_ref[...], preferred_element_type=jnp.float32)
        p_ref[...] = p

        @pl.when(pl.program_id(0) == 0)
        def _():
            s_ref[...] = jnp.zeros_like(s_ref)
            q_ref[...] = jnp.zeros_like(q_ref)

        s_ref[...] += jnp.sum(p, axis=0, keepdims=True)
        q_ref[...] += jnp.sum(p * p, axis=0, keepdims=True)

    return pl.pallas_call(
        kern,
        grid=(grid,),
        in_specs=[pl.BlockSpec((BLK, f), lambda i: (i, 0)),
                  pl.BlockSpec((f, h), lambda i: (0, 0))],
        out_specs=[pl.BlockSpec((BLK, h), lambda i: (i, 0)),
                   pl.BlockSpec((1, h), lambda i: (0, 0)),
                   pl.BlockSpec((1, h), lambda i: (0, 0))],
        out_shape=[jax.ShapeDtypeStruct((r, h), jnp.float32),
                   jax.ShapeDtypeStruct((1, h), jnp.float32),
                   jax.ShapeDtypeStruct((1, h), jnp.float32)],
    )(x, w)


def _bn_relu_mm_stats(p, s1, q1, g1, be1, w2):
    r, h = p.shape
    d = w2.shape[1]
    grid = r // BLK

    def kern(p_ref, s_ref, q_ref, g_ref, be_ref, w_ref, out_ref, s2_ref, q2_ref):
        n = jnp.float32(r)
        m = s_ref[...] / n
        var = q_ref[...] / n - m * m
        istd = g_ref[...] / jnp.sqrt(var + 1e-5)
        a = (p_ref[...] - m) * istd + be_ref[...]
        a = jnp.maximum(a, 0.0)
        q = jnp.dot(a, w_ref[...], preferred_element_type=jnp.float32)
        out_ref[...] = q

        @pl.when(pl.program_id(0) == 0)
        def _():
            s2_ref[...] = jnp.zeros_like(s2_ref)
            q2_ref[...] = jnp.zeros_like(q2_ref)

        s2_ref[...] += jnp.sum(q, axis=0, keepdims=True)
        q2_ref[...] += jnp.sum(q * q, axis=0, keepdims=True)

    return pl.pallas_call(
        kern,
        grid=(grid,),
        in_specs=[pl.BlockSpec((BLK, h), lambda i: (i, 0)),
                  pl.BlockSpec((1, h), lambda i: (0, 0)),
                  pl.BlockSpec((1, h), lambda i: (0, 0)),
                  pl.BlockSpec((1, h), lambda i: (0, 0)),
                  pl.BlockSpec((1, h), lambda i: (0, 0)),
                  pl.BlockSpec((h, d), lambda i: (0, 0))],
        out_specs=[pl.BlockSpec((BLK, d), lambda i: (i, 0)),
                   pl.BlockSpec((1, d), lambda i: (0, 0)),
                   pl.BlockSpec((1, d), lambda i: (0, 0))],
        out_shape=[jax.ShapeDtypeStruct((r, d), jnp.float32),
                   jax.ShapeDtypeStruct((1, d), jnp.float32),
                   jax.ShapeDtypeStruct((1, d), jnp.float32)],
    )(p, s1.reshape(1, h), q1.reshape(1, h), g1.reshape(1, h),
      be1.reshape(1, h), w2)


def _finalize(q, s2, q2, g2, be2, e0, e1, e2, e3):
    r, d = q.shape
    grid = r // BLK

    def kern(q_ref, s_ref, qq_ref, g_ref, be_ref, a_ref, b_ref, c_ref, d_ref,
             out_ref):
        n = jnp.float32(r)
        m = s_ref[...] / n
        var = qq_ref[...] / n - m * m
        istd = g_ref[...] / jnp.sqrt(var + 1e-5)
        feat = (q_ref[...] - m) * istd + be_ref[...]
        fin = 0.25 * (a_ref[...] + b_ref[...] + c_ref[...] + d_ref[...])
        y = (1.0 - FW) * fin + FW * feat
        nrm = jnp.sqrt(jnp.sum(y * y, axis=1, keepdims=True))
        out_ref[...] = y / jnp.maximum(nrm, 1e-12)

    return pl.pallas_call(
        kern,
        grid=(grid,),
        in_specs=[pl.BlockSpec((BLK, d), lambda i: (i, 0)),
                  pl.BlockSpec((1, d), lambda i: (0, 0)),
                  pl.BlockSpec((1, d), lambda i: (0, 0)),
                  pl.BlockSpec((1, d), lambda i: (0, 0)),
                  pl.BlockSpec((1, d), lambda i: (0, 0)),
                  pl.BlockSpec((BLK, d), lambda i: (i, 0)),
                  pl.BlockSpec((BLK, d), lambda i: (i, 0)),
                  pl.BlockSpec((BLK, d), lambda i: (i, 0)),
                  pl.BlockSpec((BLK, d), lambda i: (i, 0))],
        out_specs=pl.BlockSpec((BLK, d), lambda i: (i, 0)),
        out_shape=jax.ShapeDtypeStruct((r, d), jnp.float32),
    )(q, s2.reshape(1, d), q2.reshape(1, d), g2.reshape(1, d),
      be2.reshape(1, d), e0, e1, e2, e3)


def _mlp_side(x, w1, g1, be1, w2, g2, be2, e0, e1, e2, e3):
    p, s1, q1 = _mm_stats(x, w1)
    q, s2, q2 = _bn_relu_mm_stats(p, s1, q1, g1, be1, w2)
    return _finalize(q, s2, q2, g2, be2, e0, e1, e2, e3)


def kernel(graph_indices, graph_values, user_features, item_features,
           user_emb, item_emb,
           u_W1, u_b1, u_g1, u_be1, u_W2, u_b2, u_g2, u_be2,
           i_W1, i_b1, i_g1, i_be1, i_W2, i_b2, i_g2, i_be2):
    row = graph_indices[0].astype(jnp.int32)
    col = graph_indices[1].astype(jnp.int32)
    val = graph_values.astype(jnp.float32)

    shift = jnp.int32(HALF_PAD - N_USERS)
    rowp = row + shift * (row >= N_USERS).astype(jnp.int32)
    colp = col + shift * (col >= N_USERS).astype(jnp.int32)
    pad = E_PAD - E
    rowp = jnp.pad(rowp, (0, pad))
    colp = jnp.pad(colp, (0, pad))
    valp = jnp.pad(val, (0, pad))
    lane = jnp.arange(E_PAD, dtype=jnp.int32) % CHUNK
    dead = N_USERS + lane
    cores = []
    core_vals = []
    for cc in (0, 1):
        lo = cc * HALF_PAD
        inh = (rowp >= lo) & (rowp < lo + HALF_PAD)
        lr = jnp.where(inh, rowp - lo, dead)
        lv = jnp.where(inh, valp, 0.0)
        cores.append(jnp.stack(
            [colp.reshape(-1, CHUNK), lr.reshape(-1, CHUNK)], axis=1))
        core_vals.append(lv.reshape(-1, CHUNK))
    edges = jnp.stack(cores, axis=0)
    evals = jnp.stack(core_vals, axis=0)

    zpad = jnp.zeros((HALF_PAD - N_USERS, D), jnp.float32)
    emb = jnp.concatenate([user_emb, zpad, item_emb, zpad], axis=0)

    l1 = _sc_propagate(emb, edges, evals)
    l2 = _sc_propagate(l1, edges, evals)
    l3 = _sc_propagate(l2, edges, evals)

    u_sl = slice(0, N_USERS)
    i_sl = slice(HALF_PAD, HALF_PAD + N_ITEMS)
    user_final = _mlp_side(user_features, u_W1, u_g1, u_be1, u_W2, u_g2, u_be2,
                           emb[u_sl], l1[u_sl], l2[u_sl], l3[u_sl])
    item_final = _mlp_side(item_features, i_W1, i_g1, i_be1, i_W2, i_g2, i_be2,
                           emb[i_sl], l1[i_sl], l2[i_sl], l3[i_sl])
    return (user_final, item_final)


# speedup vs baseline: 1.0045x; 1.0045x over previous
import functools

import jax
import jax.numpy as jnp
from jax import lax
from jax.experimental import pallas as pl
from jax.experimental.pallas import tpu as pltpu
from jax.experimental.pallas import tpu_sc as plsc

N_USERS = 25000
N_ITEMS = 25000
D = 64
E = 800000
FW = 0.3

HALF_PAD = 25600
NPAD = 2 * HALF_PAD
TILES = 16
CHUNK = 128
KCH = 391
E_PAD = TILES * KCH * CHUNK
ROWS_PER_TILE = HALF_PAD // TILES
ZB = 32
NZB = ROWS_PER_TILE // ZB

BLK = 1000


def _sc_propagate(cur, col2d, row2d, val2d):
    mesh = plsc.VectorSubcoreMesh(core_axis_name="c", subcore_axis_name="s")

    @functools.partial(
        pl.kernel,
        out_type=jax.ShapeDtypeStruct((NPAD, D), jnp.float32),
        mesh=mesh,
        compiler_params=pltpu.CompilerParams(use_tc_tiling_on_sc=False),
        scratch_types=[
            pltpu.VMEM_SHARED((HALF_PAD, D), jnp.float32),
            pltpu.VMEM((4, 2, CHUNK), jnp.int32),
            pltpu.VMEM((4, CHUNK), jnp.float32),
            pltpu.VMEM((3, CHUNK, D), jnp.float32),
            pltpu.VMEM((ZB, D), jnp.float32),
            pltpu.SemaphoreType.DMA((4,)),
            pltpu.SemaphoreType.DMA((4,)),
            pltpu.SemaphoreType.DMA((4,)),
            pltpu.SemaphoreType.DMA((3,)),
            pltpu.SemaphoreType.DMA((3,)),
            pltpu.SemaphoreType.DMA,
        ],
    )
    def k(cur_hbm, col_hbm, row_hbm, val_hbm, out_hbm,
          acc, ebuf, vbuf, rows, zblk, sem_e, sem_r, sem_v, sem_g, sem_s,
          sem_z):
        c = lax.axis_index("c")
        s = lax.axis_index("s")
        lo = c * HALF_PAD
        rbase = s * ROWS_PER_TILE
        base_ch = s * KCH

        def zz(i, carry):
            for j in range(D // 16):
                zblk[i, pl.ds(j * 16, 16)] = jnp.zeros((16,), jnp.float32)
            return carry
        lax.fori_loop(0, ZB, zz, 0)

        def zacc(b, carry):
            pltpu.async_copy(zblk, acc.at[pl.ds(rbase + b * ZB, ZB)], sem_z)
            return carry
        lax.fori_loop(0, NZB, zacc, 0)

        def zdrain(b, carry):
            pltpu.make_async_copy(
                zblk, acc.at[pl.ds(rbase + b * ZB, ZB)], sem_z).wait()
            return carry
        lax.fori_loop(0, NZB, zdrain, 0)
        plsc.subcore_barrier()

        def col_desc(kk):
            t = lax.rem(kk, 4)
            return pltpu.make_async_copy(
                col_hbm.at[base_ch + kk], ebuf.at[t, 0], sem_e.at[t])

        def row_desc(kk):
            t = lax.rem(kk, 4)
            return pltpu.make_async_copy(
                row_hbm.at[base_ch + kk], ebuf.at[t, 1], sem_r.at[t])

        def val_desc(kk):
            t = lax.rem(kk, 4)
            return pltpu.make_async_copy(
                val_hbm.at[base_ch + kk], vbuf.at[t], sem_v.at[t])

        def gather_desc(kk):
            t = lax.rem(kk, 4)
            b = lax.rem(kk, 3)
            return pltpu.make_async_copy(
                cur_hbm.at[ebuf.at[t, 0]], rows.at[b], sem_g.at[b])

        def scatter_desc(kk):
            t = lax.rem(kk, 4)
            b = lax.rem(kk, 3)
            return pltpu.make_async_copy(
                rows.at[b], acc.at[ebuf.at[t, 1]], sem_s.at[b])

        dnums = lax.GatherDimensionNumbers(
            offset_dims=(), collapsed_slice_dims=(0,), start_index_map=(0,))
        shift = jnp.int32(HALF_PAD - N_USERS)

        def remap_col(kk):
            t = lax.rem(kk, 4)

            @plsc.parallel_loop(0, CHUNK // 16, unroll=2)
            def _(g):
                sl = pl.ds(g * 16, 16)
                cv = ebuf[t, 0, sl]
                ebuf[t, 0, sl] = jnp.where(cv >= N_USERS, cv + shift, cv)

        def mask_row(kk):
            t = lax.rem(kk, 4)
            iota = lax.iota(jnp.int32, 16)

            @plsc.parallel_loop(0, CHUNK // 16, unroll=2)
            def _(g):
                sl = pl.ds(g * 16, 16)
                rv = ebuf[t, 1, sl]
                rv = jnp.where(rv >= N_USERS, rv + shift, rv)
                inh = (rv >= lo) & (rv < lo + HALF_PAD)
                dead = (N_USERS + g * 16) + iota
                ebuf[t, 1, sl] = jnp.where(inh, rv - lo, dead)
                vv = vbuf[t, sl]
                vbuf[t, sl] = jnp.where(
                    inh, vv, jnp.zeros((16,), jnp.float32))

        for j in range(2):
            col_desc(j).start()
            row_desc(j).start()
            val_desc(j).start()
        col_desc(0).wait()
        remap_col(0)
        gather_desc(0).start()

        def chunk_body(kk, carry):
            t = lax.rem(kk, 4)
            b = lax.rem(kk, 3)

            @pl.when(kk >= 2)
            def _():
                scatter_desc(kk - 2).wait()

            @pl.when(kk + 1 < KCH)
            def _():
                col_desc(kk + 1).wait()
                remap_col(kk + 1)
                gather_desc(kk + 1).start()

            @pl.when(kk + 2 < KCH)
            def _():
                col_desc(kk + 2).start()
                row_desc(kk + 2).start()
                val_desc(kk + 2).start()
            val_desc(kk).wait()
            row_desc(kk).wait()
            mask_row(kk)
            gather_desc(kk).wait()

            @plsc.parallel_loop(0, CHUNK // 16, unroll=2)
            def scale_group(g):
                vbase = vbuf[t, pl.ds(g * 16, 16)]
                for i in range(16):
                    v16 = lax.gather(
                        vbase, jnp.full((16, 1), i, jnp.int32), dnums, (1,),
                        mode=lax.GatherScatterMode.PROMISE_IN_BOUNDS)
                    e = g * 16 + i
                    for j in range(D // 16):
                        sl = pl.ds(j * 16, 16)
                        rows[b, e, sl] = rows[b, e, sl] * v16

            scatter_desc(kk).start(add=True)
            return carry
        lax.fori_loop(0, KCH, chunk_body, 0)
        scatter_desc(KCH - 2).wait()
        scatter_desc(KCH - 1).wait()
        plsc.subcore_barrier()

        def wb(b, carry):
            st = rbase + b * ZB
            pltpu.async_copy(
                acc.at[pl.ds(st, ZB)], out_hbm.at[pl.ds(lo + st, ZB)], sem_z)
            return carry
        lax.fori_loop(0, NZB, wb, 0)

        def wb_drain(b, carry):
            st = rbase + b * ZB
            pltpu.make_async_copy(
                acc.at[pl.ds(st, ZB)], out_hbm.at[pl.ds(lo + st, ZB)],
                sem_z).wait()
            return carry
        lax.fori_loop(0, NZB, wb_drain, 0)

    return k(cur, col2d, row2d, val2d)


def _mm_stats(x, w):
    r, f = x.shape
    h = w.shape[1]
    grid = r // BLK

    def kern(x_ref, w_ref, p_ref, s_ref, q_ref):
        p = jnp.dot(x_ref[...], w_ref[...], preferred_element_type=jnp.float32)
        p_ref[...] = p

        @pl.when(pl.program_id(0) == 0)
        def _():
            s_ref[...] = jnp.zeros_like(s_ref)
            q_ref[...] = jnp.zeros_like(q_ref)

        s_ref[...] += jnp.sum(p, axis=0, keepdims=True)
        q_ref[...] += jnp.sum(p * p, axis=0, keepdims=True)

    return pl.pallas_call(
        kern,
        grid=(grid,),
        in_specs=[pl.BlockSpec((BLK, f), lambda i: (i, 0)),
                  pl.BlockSpec((f, h), lambda i: (0, 0))],
        out_specs=[pl.BlockSpec((BLK, h), lambda i: (i, 0)),
                   pl.BlockSpec((1, h), lambda i: (0, 0)),
                   pl.BlockSpec((1, h), lambda i: (0, 0))],
        out_shape=[jax.ShapeDtypeStruct((r, h), jnp.float32),
                   jax.ShapeDtypeStruct((1, h), jnp.float32),
                   jax.ShapeDtypeStruct((1, h), jnp.float32)],
    )(x, w)


def _bn_relu_mm_stats(p, s1, q1, g1, be1, w2):
    r, h = p.shape
    d = w2.shape[1]
    grid = r // BLK

    def kern(p_ref, s_ref, q_ref, g_ref, be_ref, w_ref, out_ref, s2_ref, q2_ref):
        n = jnp.float32(r)
        m = s_ref[...] / n
        var = q_ref[...] / n - m * m
        istd = g_ref[...] / jnp.sqrt(var + 1e-5)
        a = (p_ref[...] - m) * istd + be_ref[...]
        a = jnp.maximum(a, 0.0)
        q = jnp.dot(a, w_ref[...], preferred_element_type=jnp.float32)
        out_ref[...] = q

        @pl.when(pl.program_id(0) == 0)
        def _():
            s2_ref[...] = jnp.zeros_like(s2_ref)
            q2_ref[...] = jnp.zeros_like(q2_ref)

        s2_ref[...] += jnp.sum(q, axis=0, keepdims=True)
        q2_ref[...] += jnp.sum(q * q, axis=0, keepdims=True)

    return pl.pallas_call(
        kern,
        grid=(grid,),
        in_specs=[pl.BlockSpec((BLK, h), lambda i: (i, 0)),
                  pl.BlockSpec((1, h), lambda i: (0, 0)),
                  pl.BlockSpec((1, h), lambda i: (0, 0)),
                  pl.BlockSpec((1, h), lambda i: (0, 0)),
                  pl.BlockSpec((1, h), lambda i: (0, 0)),
                  pl.BlockSpec((h, d), lambda i: (0, 0))],
        out_specs=[pl.BlockSpec((BLK, d), lambda i: (i, 0)),
                   pl.BlockSpec((1, d), lambda i: (0, 0)),
                   pl.BlockSpec((1, d), lambda i: (0, 0))],
        out_shape=[jax.ShapeDtypeStruct((r, d), jnp.float32),
                   jax.ShapeDtypeStruct((1, d), jnp.float32),
                   jax.ShapeDtypeStruct((1, d), jnp.float32)],
    )(p, s1.reshape(1, h), q1.reshape(1, h), g1.reshape(1, h),
      be1.reshape(1, h), w2)


def _finalize(q, s2, q2, g2, be2, e0, e1, e2, e3):
    r, d = q.shape
    grid = r // BLK

    def kern(q_ref, s_ref, qq_ref, g_ref, be_ref, a_ref, b_ref, c_ref, d_ref,
             out_ref):
        n = jnp.float32(r)
        m = s_ref[...] / n
        var = qq_ref[...] / n - m * m
        istd = g_ref[...] / jnp.sqrt(var + 1e-5)
        feat = (q_ref[...] - m) * istd + be_ref[...]
        fin = 0.25 * (a_ref[...] + b_ref[...] + c_ref[...] + d_ref[...])
        y = (1.0 - FW) * fin + FW * feat
        nrm = jnp.sqrt(jnp.sum(y * y, axis=1, keepdims=True))
        out_ref[...] = y / jnp.maximum(nrm, 1e-12)

    return pl.pallas_call(
        kern,
        grid=(grid,),
        in_specs=[pl.BlockSpec((BLK, d), lambda i: (i, 0)),
                  pl.BlockSpec((1, d), lambda i: (0, 0)),
                  pl.BlockSpec((1, d), lambda i: (0, 0)),
                  pl.BlockSpec((1, d), lambda i: (0, 0)),
                  pl.BlockSpec((1, d), lambda i: (0, 0)),
                  pl.BlockSpec((BLK, d), lambda i: (i, 0)),
                  pl.BlockSpec((BLK, d), lambda i: (i, 0)),
                  pl.BlockSpec((BLK, d), lambda i: (i, 0)),
                  pl.BlockSpec((BLK, d), lambda i: (i, 0))],
        out_specs=pl.BlockSpec((BLK, d), lambda i: (i, 0)),
        out_shape=jax.ShapeDtypeStruct((r, d), jnp.float32),
    )(q, s2.reshape(1, d), q2.reshape(1, d), g2.reshape(1, d),
      be2.reshape(1, d), e0, e1, e2, e3)


def _mlp_side(x, w1, g1, be1, w2, g2, be2, e0, e1, e2, e3):
    p, s1, q1 = _mm_stats(x, w1)
    q, s2, q2 = _bn_relu_mm_stats(p, s1, q1, g1, be1, w2)
    return _finalize(q, s2, q2, g2, be2, e0, e1, e2, e3)


def kernel(graph_indices, graph_values, user_features, item_features,
           user_emb, item_emb,
           u_W1, u_b1, u_g1, u_be1, u_W2, u_b2, u_g2, u_be2,
           i_W1, i_b1, i_g1, i_be1, i_W2, i_b2, i_g2, i_be2):
    row = graph_indices[0].astype(jnp.int32)
    col = graph_indices[1].astype(jnp.int32)
    val = graph_values.astype(jnp.float32)

    pad = E_PAD - E
    colp = jnp.pad(col, (0, pad)).reshape(-1, CHUNK)
    rowp = jnp.pad(row, (0, pad)).reshape(-1, CHUNK)
    valp = jnp.pad(val, (0, pad)).reshape(-1, CHUNK)

    zpad = jnp.zeros((HALF_PAD - N_USERS, D), jnp.float32)
    emb = jnp.concatenate([user_emb, zpad, item_emb, zpad], axis=0)

    l1 = _sc_propagate(emb, colp, rowp, valp)
    l2 = _sc_propagate(l1, colp, rowp, valp)
    l3 = _sc_propagate(l2, colp, rowp, valp)

    u_sl = slice(0, N_USERS)
    i_sl = slice(HALF_PAD, HALF_PAD + N_ITEMS)
    user_final = _mlp_side(user_features, u_W1, u_g1, u_be1, u_W2, u_g2, u_be2,
                           emb[u_sl], l1[u_sl], l2[u_sl], l3[u_sl])
    item_final = _mlp_side(item_features, i_W1, i_g1, i_be1, i_W2, i_g2, i_be2,
                           emb[i_sl], l1[i_sl], l2[i_sl], l3[i_sl])
    return (user_final, item_final)
